# baseline (speedup 1.0000x reference)
import functools

import jax
import jax.numpy as jnp
from jax.experimental import pallas as pl
from jax.experimental.pallas import tpu as pltpu

_EMB = 1024
_HEADS = 16
_T = 2048
_FF = 4

_BQ = 512
_BK = 1024
_NQ = _T // _BQ
_NK = _T // _BK

_VMEM_LIMIT = 60 * 1024 * 1024


def _qkv_body(x_ref, wq_ref, wk_ref, wv_ref, q_ref, k_ref, v_ref):
    x = x_ref[...]
    wq = wq_ref[...].astype(jnp.bfloat16)
    qo = jax.lax.dot(x, wq, preferred_element_type=jnp.float32)
    q_ref[0] = (qo * (1.0 / 32.0)).astype(jnp.bfloat16)
    wk = wk_ref[...].astype(jnp.bfloat16)
    k_ref[0] = jax.lax.dot(
        x, wk, preferred_element_type=jnp.float32
    ).astype(jnp.bfloat16)
    wv = wv_ref[...].astype(jnp.bfloat16)
    v_ref[0] = jax.lax.dot(
        x, wv, preferred_element_type=jnp.float32
    ).astype(jnp.bfloat16)


def _layernorm(t, g, b):
    m = jnp.mean(t, axis=1, keepdims=True)
    c = t - m
    v = jnp.mean(c * c, axis=1, keepdims=True)
    return c * jax.lax.rsqrt(v + 1e-5) * g + b


def _attn_proj_body(
    q_ref, k_ref, v_ref, wu_ref, x_ref, bu_ref, g_ref, be_ref,
    out_ref, acc0_ref, m0_ref, l0_ref, acc1_ref, m1_ref, l1_ref, pacc_ref,
):
    h = pl.program_id(0)
    j = pl.program_id(1)
    kk = pl.program_id(2)

    chains = ((0, acc0_ref, m0_ref, l0_ref), (1, acc1_ref, m1_ref, l1_ref))
    for c, acc_ref, m_ref, l_ref in chains:
        iq = 2 * j + c

        def _flash_update(masked, first, c, acc_ref, m_ref, l_ref):
            q = q_ref[0, c * _BQ:(c + 1) * _BQ, :]
            k = k_ref[0]
            st = jax.lax.dot_general(
                k, q, (((1,), (1,)), ((), ())),
                preferred_element_type=jnp.float32,
            )
            if masked:
                keys = jax.lax.broadcasted_iota(jnp.int32, (_BK, _BQ), 0)
                rows = jax.lax.broadcasted_iota(jnp.int32, (_BK, _BQ), 1)
                st = jnp.where(keys <= rows + c * _BQ, st, -jnp.inf)
            if first:
                m_new = jnp.max(st, axis=0, keepdims=True)
                pt = jnp.exp(st - m_new)
                l_ref[...] = jnp.sum(pt, axis=0, keepdims=True)
                acc_ref[...] = jax.lax.dot_general(
                    v_ref[0], pt.astype(jnp.bfloat16),
                    (((0,), (0,)), ((), ())),
                    preferred_element_type=jnp.float32,
                )
            else:
                m_prev = m_ref[...]
                m_new = jnp.maximum(m_prev, jnp.max(st, axis=0, keepdims=True))
                alpha = jnp.exp(m_prev - m_new)
                pt = jnp.exp(st - m_new)
                l_ref[...] = l_ref[...] * alpha + jnp.sum(
                    pt, axis=0, keepdims=True
                )
                acc_ref[...] = acc_ref[...] * alpha + jax.lax.dot_general(
                    v_ref[0], pt.astype(jnp.bfloat16),
                    (((0,), (0,)), ((), ())),
                    preferred_element_type=jnp.float32,
                )
            m_ref[...] = m_new

        @pl.when((kk == 0) & (j == 0))
        def _upd_first_diag(c=c, acc_ref=acc_ref, m_ref=m_ref, l_ref=l_ref):
            _flash_update(True, True, c, acc_ref, m_ref, l_ref)

        @pl.when((kk == 0) & (j > 0))
        def _upd_first(c=c, acc_ref=acc_ref, m_ref=m_ref, l_ref=l_ref):
            _flash_update(False, True, c, acc_ref, m_ref, l_ref)

        @pl.when((kk == j) & (kk > 0))
        def _upd_diag(c=c, acc_ref=acc_ref, m_ref=m_ref, l_ref=l_ref):
            _flash_update(True, False, c, acc_ref, m_ref, l_ref)

        @pl.when(kk == j)
        def _proj(c=c, iq=iq, acc_ref=acc_ref, l_ref=l_ref):
            o = (acc_ref[...] / l_ref[...]).astype(jnp.bfloat16)
            wu = wu_ref[...].astype(jnp.bfloat16)
            part = jax.lax.dot_general(
                o, wu, (((0,), (0,)), ((), ())),
                preferred_element_type=jnp.float32,
            )
            base = pl.multiple_of(iq * _BQ, _BQ)

            @pl.when(h == 0)
            def _first_head():
                pacc_ref[pl.ds(base, _BQ), :] = part

            @pl.when(h > 0)
            def _other_heads():
                pacc_ref[pl.ds(base, _BQ), :] = (
                    pacc_ref[pl.ds(base, _BQ), :] + part
                )

            @pl.when(h == _HEADS - 1)
            def _finish():
                t = (
                    pacc_ref[pl.ds(base, _BQ), :]
                    + bu_ref[...]
                    + x_ref[c * _BQ:(c + 1) * _BQ, :]
                )
                out_ref[pl.ds(base, _BQ), :] = _layernorm(
                    t, g_ref[...], be_ref[...]
                )


def _ff_ln_body(x_ref, w1_ref, b1_ref, w2_ref, b2_ref, g_ref, be_ref, out_ref):
    x = x_ref[...]
    w1 = w1_ref[...].astype(jnp.bfloat16)
    hid = jax.lax.dot(
        x.astype(jnp.bfloat16), w1, preferred_element_type=jnp.float32
    )
    hid = jnp.maximum(hid + b1_ref[...], 0.0)
    w2 = w2_ref[...].astype(jnp.bfloat16)
    f = jax.lax.dot(
        hid.astype(jnp.bfloat16), w2, preferred_element_type=jnp.float32
    )
    t = f + b2_ref[...] + x
    out_ref[...] = _layernorm(t, g_ref[...], be_ref[...])


def kernel(x, Wq, Wk, Wv, Wu, bu, g1, be1, g2, be2, W1, b1, W2, b2):
    b, t, e = x.shape
    x2d = x.reshape(t, e)
    xb = x2d.astype(jnp.bfloat16)

    cp = lambda sem: pltpu.CompilerParams(
        dimension_semantics=sem, vmem_limit_bytes=_VMEM_LIMIT
    )

    q, k, v = pl.pallas_call(
        _qkv_body,
        grid=(_HEADS,),
        in_specs=[
            pl.BlockSpec((_T, _EMB), lambda h: (0, 0)),
            pl.BlockSpec((_EMB, _EMB), lambda h: (0, h)),
            pl.BlockSpec((_EMB, _EMB), lambda h: (0, h)),
            pl.BlockSpec((_EMB, _EMB), lambda h: (0, h)),
        ],
        out_specs=[
            pl.BlockSpec((1, _T, _EMB), lambda h: (h, 0, 0)),
            pl.BlockSpec((1, _T, _EMB), lambda h: (h, 0, 0)),
            pl.BlockSpec((1, _T, _EMB), lambda h: (h, 0, 0)),
        ],
        out_shape=[
            jax.ShapeDtypeStruct((_HEADS, _T, _EMB), jnp.bfloat16),
            jax.ShapeDtypeStruct((_HEADS, _T, _EMB), jnp.bfloat16),
            jax.ShapeDtypeStruct((_HEADS, _T, _EMB), jnp.bfloat16),
        ],
        compiler_params=cp(("arbitrary",)),
    )(xb, Wq, Wk, Wv)

    x1 = pl.pallas_call(
        _attn_proj_body,
        grid=(_HEADS, _NQ // 2, _NK),
        in_specs=[
            pl.BlockSpec((1, 2 * _BQ, _EMB), lambda h, j, kk: (h, j, 0)),
            pl.BlockSpec(
                (1, _BK, _EMB),
                lambda h, j, kk: (h, jnp.minimum(kk, j), 0),
            ),
            pl.BlockSpec(
                (1, _BK, _EMB),
                lambda h, j, kk: (h, jnp.minimum(kk, j), 0),
            ),
            pl.BlockSpec((_EMB, _EMB), lambda h, j, ik: (h, 0)),
            pl.BlockSpec((2 * _BQ, _EMB), lambda h, j, kk: (j, 0)),
            pl.BlockSpec((1, _EMB), lambda h, j, kk: (0, 0)),
            pl.BlockSpec((1, _EMB), lambda h, j, kk: (0, 0)),
            pl.BlockSpec((1, _EMB), lambda h, j, kk: (0, 0)),
        ],
        out_specs=pl.BlockSpec((_T, _EMB), lambda h, j, kk: (0, 0)),
        out_shape=jax.ShapeDtypeStruct((_T, _EMB), jnp.float32),
        scratch_shapes=[
            pltpu.VMEM((_EMB, _BQ), jnp.float32),
            pltpu.VMEM((1, _BQ), jnp.float32),
            pltpu.VMEM((1, _BQ), jnp.float32),
            pltpu.VMEM((_EMB, _BQ), jnp.float32),
            pltpu.VMEM((1, _BQ), jnp.float32),
            pltpu.VMEM((1, _BQ), jnp.float32),
            pltpu.VMEM((_T, _EMB), jnp.float32),
        ],
        compiler_params=cp(("arbitrary", "arbitrary", "arbitrary")),
    )(
        q,
        k,
        v,
        Wu,
        x2d,
        bu.reshape(1, _EMB),
        g1.reshape(1, _EMB),
        be1.reshape(1, _EMB),
    )

    br = 512
    x2 = pl.pallas_call(
        _ff_ln_body,
        grid=(_T // br,),
        in_specs=[
            pl.BlockSpec((br, _EMB), lambda i: (i, 0)),
            pl.BlockSpec((_EMB, _FF * _EMB), lambda i: (0, 0)),
            pl.BlockSpec((1, _FF * _EMB), lambda i: (0, 0)),
            pl.BlockSpec((_FF * _EMB, _EMB), lambda i: (0, 0)),
            pl.BlockSpec((1, _EMB), lambda i: (0, 0)),
            pl.BlockSpec((1, _EMB), lambda i: (0, 0)),
            pl.BlockSpec((1, _EMB), lambda i: (0, 0)),
        ],
        out_specs=pl.BlockSpec((br, _EMB), lambda i: (i, 0)),
        out_shape=jax.ShapeDtypeStruct((_T, _EMB), jnp.float32),
        compiler_params=cp(("arbitrary",)),
    )(
        x1,
        W1,
        b1.reshape(1, _FF * _EMB),
        W2,
        b2.reshape(1, _EMB),
        g2.reshape(1, _EMB),
        be2.reshape(1, _EMB),
    )

    return x2.reshape(b, t, e)


# speedup vs baseline: 1.0619x; 1.0619x over previous
import functools

import jax
import jax.numpy as jnp
from jax.experimental import pallas as pl
from jax.experimental.pallas import tpu as pltpu

_EMB = 1024
_HEADS = 16
_T = 2048
_FF = 4

_BQ = 512
_BK = 1024
_BQ2 = 1024
_NQ = _T // _BQ
_NK = _T // _BK

_VMEM_LIMIT = 60 * 1024 * 1024


def _qkv_body(x_ref, wq_ref, wk_ref, wv_ref, q_ref, k_ref, v_ref):
    x = x_ref[...]
    wq = wq_ref[...].astype(jnp.bfloat16)
    qo = jax.lax.dot(x, wq, preferred_element_type=jnp.float32)
    q_ref[0] = (qo * (1.0 / 32.0)).astype(jnp.bfloat16)
    wk = wk_ref[...].astype(jnp.bfloat16)
    k_ref[0] = jax.lax.dot(
        x, wk, preferred_element_type=jnp.float32
    ).astype(jnp.bfloat16)
    wv = wv_ref[...].astype(jnp.bfloat16)
    v_ref[0] = jax.lax.dot(
        x, wv, preferred_element_type=jnp.float32
    ).astype(jnp.bfloat16)


def _layernorm(t, g, b):
    m = jnp.mean(t, axis=1, keepdims=True)
    c = t - m
    v = jnp.mean(c * c, axis=1, keepdims=True)
    return c * jax.lax.rsqrt(v + 1e-5) * g + b


def _attn_proj_body(
    q_ref, k_ref, v_ref, wu_ref, x_ref, bu_ref, g_ref, be_ref,
    out_ref, acc_ref, m_ref, l_ref, pacc_ref,
):
    h = pl.program_id(0)
    j = pl.program_id(1)
    kk = pl.program_id(2)

    def _flash_update(masked, first):
        q = q_ref[0]
        k = k_ref[0]
        st = jax.lax.dot_general(
            k, q, (((1,), (1,)), ((), ())),
            preferred_element_type=jnp.float32,
        )
        if masked:
            keys = jax.lax.broadcasted_iota(jnp.int32, (_BK, _BQ2), 0)
            rows = jax.lax.broadcasted_iota(jnp.int32, (_BK, _BQ2), 1)
            st = jnp.where(keys <= rows, st, -jnp.inf)
        if first:
            m_new = jnp.max(st, axis=0, keepdims=True)
            pt = jnp.exp(st - m_new)
            l_ref[...] = jnp.sum(pt, axis=0, keepdims=True)
            acc_ref[...] = jax.lax.dot_general(
                v_ref[0], pt.astype(jnp.bfloat16),
                (((0,), (0,)), ((), ())),
                preferred_element_type=jnp.float32,
            )
        else:
            m_prev = m_ref[...]
            m_new = jnp.maximum(m_prev, jnp.max(st, axis=0, keepdims=True))
            alpha = jnp.exp(m_prev - m_new)
            pt = jnp.exp(st - m_new)
            l_ref[...] = l_ref[...] * alpha + jnp.sum(pt, axis=0, keepdims=True)
            acc_ref[...] = acc_ref[...] * alpha + jax.lax.dot_general(
                v_ref[0], pt.astype(jnp.bfloat16),
                (((0,), (0,)), ((), ())),
                preferred_element_type=jnp.float32,
            )
        m_ref[...] = m_new

    @pl.when((kk == 0) & (j == 0))
    def _upd_first_diag():
        _flash_update(True, True)

    @pl.when((kk == 0) & (j > 0))
    def _upd_first():
        _flash_update(False, True)

    @pl.when((kk == j) & (kk > 0))
    def _upd_diag():
        _flash_update(True, False)

    @pl.when(kk == j)
    def _proj():
        o = (acc_ref[...] / l_ref[...]).astype(jnp.bfloat16)
        wu = wu_ref[...].astype(jnp.bfloat16)
        part = jax.lax.dot_general(
            o, wu, (((0,), (0,)), ((), ())),
            preferred_element_type=jnp.float32,
        )
        base = pl.multiple_of(j * _BQ2, _BQ2)

        @pl.when(h == 0)
        def _first_head():
            pacc_ref[pl.ds(base, _BQ2), :] = part

        @pl.when(h > 0)
        def _other_heads():
            pacc_ref[pl.ds(base, _BQ2), :] = (
                pacc_ref[pl.ds(base, _BQ2), :] + part
            )

        @pl.when(h == _HEADS - 1)
        def _finish():
            t = pacc_ref[pl.ds(base, _BQ2), :] + bu_ref[...] + x_ref[...]
            out_ref[pl.ds(base, _BQ2), :] = _layernorm(
                t, g_ref[...], be_ref[...]
            )


def _ff_ln_body(x_ref, w1_ref, b1_ref, w2_ref, b2_ref, g_ref, be_ref, out_ref):
    x = x_ref[...]
    w1 = w1_ref[...].astype(jnp.bfloat16)
    hid = jax.lax.dot(
        x.astype(jnp.bfloat16), w1, preferred_element_type=jnp.float32
    )
    hid = jnp.maximum(hid + b1_ref[...], 0.0)
    w2 = w2_ref[...].astype(jnp.bfloat16)
    f = jax.lax.dot(
        hid.astype(jnp.bfloat16), w2, preferred_element_type=jnp.float32
    )
    t = f + b2_ref[...] + x
    out_ref[...] = _layernorm(t, g_ref[...], be_ref[...])


def kernel(x, Wq, Wk, Wv, Wu, bu, g1, be1, g2, be2, W1, b1, W2, b2):
    b, t, e = x.shape
    x2d = x.reshape(t, e)
    xb = x2d.astype(jnp.bfloat16)

    cp = lambda sem: pltpu.CompilerParams(
        dimension_semantics=sem, vmem_limit_bytes=_VMEM_LIMIT
    )

    q, k, v = pl.pallas_call(
        _qkv_body,
        grid=(_HEADS,),
        in_specs=[
            pl.BlockSpec((_T, _EMB), lambda h: (0, 0)),
            pl.BlockSpec((_EMB, _EMB), lambda h: (0, h)),
            pl.BlockSpec((_EMB, _EMB), lambda h: (0, h)),
            pl.BlockSpec((_EMB, _EMB), lambda h: (0, h)),
        ],
        out_specs=[
            pl.BlockSpec((1, _T, _EMB), lambda h: (h, 0, 0)),
            pl.BlockSpec((1, _T, _EMB), lambda h: (h, 0, 0)),
            pl.BlockSpec((1, _T, _EMB), lambda h: (h, 0, 0)),
        ],
        out_shape=[
            jax.ShapeDtypeStruct((_HEADS, _T, _EMB), jnp.bfloat16),
            jax.ShapeDtypeStruct((_HEADS, _T, _EMB), jnp.bfloat16),
            jax.ShapeDtypeStruct((_HEADS, _T, _EMB), jnp.bfloat16),
        ],
        compiler_params=cp(("arbitrary",)),
    )(xb, Wq, Wk, Wv)

    x1 = pl.pallas_call(
        _attn_proj_body,
        grid=(_HEADS, _NQ // 2, _NK),
        in_specs=[
            pl.BlockSpec((1, 2 * _BQ, _EMB), lambda h, j, kk: (h, j, 0)),
            pl.BlockSpec(
                (1, _BK, _EMB),
                lambda h, j, kk: (h, jnp.minimum(kk, j), 0),
            ),
            pl.BlockSpec(
                (1, _BK, _EMB),
                lambda h, j, kk: (h, jnp.minimum(kk, j), 0),
            ),
            pl.BlockSpec((_EMB, _EMB), lambda h, j, ik: (h, 0)),
            pl.BlockSpec((2 * _BQ, _EMB), lambda h, j, kk: (j, 0)),
            pl.BlockSpec((1, _EMB), lambda h, j, kk: (0, 0)),
            pl.BlockSpec((1, _EMB), lambda h, j, kk: (0, 0)),
            pl.BlockSpec((1, _EMB), lambda h, j, kk: (0, 0)),
        ],
        out_specs=pl.BlockSpec((_T, _EMB), lambda h, j, kk: (0, 0)),
        out_shape=jax.ShapeDtypeStruct((_T, _EMB), jnp.float32),
        scratch_shapes=[
            pltpu.VMEM((_EMB, _BQ2), jnp.float32),
            pltpu.VMEM((1, _BQ2), jnp.float32),
            pltpu.VMEM((1, _BQ2), jnp.float32),
            pltpu.VMEM((_T, _EMB), jnp.float32),
        ],
        compiler_params=cp(("arbitrary", "arbitrary", "arbitrary")),
    )(
        q,
        k,
        v,
        Wu,
        x2d,
        bu.reshape(1, _EMB),
        g1.reshape(1, _EMB),
        be1.reshape(1, _EMB),
    )

    br = 512
    x2 = pl.pallas_call(
        _ff_ln_body,
        grid=(_T // br,),
        in_specs=[
            pl.BlockSpec((br, _EMB), lambda i: (i, 0)),
            pl.BlockSpec((_EMB, _FF * _EMB), lambda i: (0, 0)),
            pl.BlockSpec((1, _FF * _EMB), lambda i: (0, 0)),
            pl.BlockSpec((_FF * _EMB, _EMB), lambda i: (0, 0)),
            pl.BlockSpec((1, _EMB), lambda i: (0, 0)),
            pl.BlockSpec((1, _EMB), lambda i: (0, 0)),
            pl.BlockSpec((1, _EMB), lambda i: (0, 0)),
        ],
        out_specs=pl.BlockSpec((br, _EMB), lambda i: (i, 0)),
        out_shape=jax.ShapeDtypeStruct((_T, _EMB), jnp.float32),
        compiler_params=cp(("arbitrary",)),
    )(
        x1,
        W1,
        b1.reshape(1, _FF * _EMB),
        W2,
        b2.reshape(1, _EMB),
        g2.reshape(1, _EMB),
        be2.reshape(1, _EMB),
    )

    return x2.reshape(b, t, e)
